# Initial kernel scaffold; baseline (speedup 1.0000x reference)
import jax
import jax.numpy as jnp
from jax.experimental import pallas as pl


def kernel(inputs, token_table, pos_table):
    raise NotImplementedError("write your pallas kernel here")


# speedup vs baseline: 3.9510x; 3.9510x over previous
import functools

import jax
import jax.numpy as jnp
from jax import lax
from jax.experimental import pallas as pl
from jax.experimental.pallas import tpu as pltpu
from jax.experimental.pallas import tpu_sc as plsc

NC, NS, L = 2, 16, 16
NW = NC * NS
B, S, D = 1024, 200, 128
SEQ_PER_W = B // NW


def _body(tok_hbm, idx_hbm, pos_hbm, out_hbm, pos_v, idx_v, rows_v, gsem):
  wid = lax.axis_index("s") * NC + lax.axis_index("c")

  pltpu.sync_copy(pos_hbm, pos_v)

  def seq_body(k, carry):
    base = (wid * SEQ_PER_W + k) * S
    pltpu.sync_copy(idx_hbm.at[pl.ds(base, S)], idx_v)
    pltpu.async_copy(tok_hbm.at[idx_v], rows_v, gsem).wait()

    def row_body(r, c):
      for j in range(D // L):
        sl = pl.ds(j * L, L)
        plsc.addupdate(rows_v.at[r, sl], pos_v[r, sl])
      return c

    lax.fori_loop(0, S, row_body, 0)
    pltpu.sync_copy(rows_v, out_hbm.at[pl.ds(base, S)])
    return carry

  lax.fori_loop(0, SEQ_PER_W, seq_body, 0)


@jax.jit
def _run(token_table, idx_flat, pos_table):
  mesh = plsc.VectorSubcoreMesh(
      core_axis_name="c", subcore_axis_name="s",
      num_cores=NC, num_subcores=NS)
  f = pl.kernel(
      _body,
      out_type=jax.ShapeDtypeStruct((B * S, D), jnp.float32),
      mesh=mesh,
      scratch_types=[
          pltpu.VMEM((S, D), jnp.float32),
          pltpu.VMEM((S,), jnp.int32),
          pltpu.VMEM((S, D), jnp.float32),
          pltpu.SemaphoreType.DMA,
      ],
  )
  return f(token_table, idx_flat, pos_table)


def kernel(inputs, token_table, pos_table):
  idx_flat = inputs.reshape(-1).astype(jnp.int32)
  out = _run(token_table, idx_flat, pos_table)
  return out.reshape(B, S, D)
